# baseline (speedup 1.0000x reference)
import functools

import jax
import jax.numpy as jnp
from jax import lax
from jax.experimental import pallas as pl
from jax.experimental.pallas import tpu as pltpu
from jax.experimental.pallas import tpu_sc as plsc

N = 10000
NP = 10240
D = 128
E = 320000
NE = 2 * E
B = 16
L = 50
LP = 64
NC = 2
NS = 16
NW = NC * NS
K = 80
CH = NE // NW
NB = CH // K
EW = E // NW
RPT = NP // NS
G = 1280 // NW

_f32 = jnp.float32
_i32 = jnp.int32

_mesh = plsc.VectorSubcoreMesh(
    core_axis_name="c", subcore_axis_name="s", num_cores=NC, num_subcores=NS)


@functools.partial(
    pl.kernel,
    out_type=jax.ShapeDtypeStruct((NC, NP), _f32),
    mesh=_mesh,
    scratch_types=[
        pltpu.VMEM((NB, K), _i32),
        pltpu.VMEM((K,), _f32),
        pltpu.VMEM((RPT,), _f32),
        pltpu.VMEM_SHARED((NP,), _f32),
    ],
)
def _deg_call(n1_hbm, out_hbm, idx_v, ones_v, zrow_v, deg_sh):
    c = lax.axis_index("c")
    s = lax.axis_index("s")
    wid = s * NC + c
    pltpu.sync_copy(n1_hbm.at[wid], idx_v)

    @pl.loop(0, K // 16)
    def _(i):
        ones_v[pl.ds(i * 16, 16)] = jnp.ones((16,), _f32)

    @pl.loop(0, RPT // 16)
    def _(i):
        zrow_v[pl.ds(i * 16, 16)] = jnp.zeros((16,), _f32)

    pltpu.sync_copy(zrow_v, deg_sh.at[pl.ds(s * RPT, RPT)])
    plsc.subcore_barrier()

    @pl.loop(0, NB)
    def _(j):
        pltpu.sync_copy(ones_v, deg_sh.at[idx_v.at[j]], add=True)

    plsc.subcore_barrier()
    pltpu.sync_copy(deg_sh.at[pl.ds(s * RPT, RPT)],
                    out_hbm.at[c, pl.ds(s * RPT, RPT)])


@functools.partial(
    pl.kernel,
    out_type=jax.ShapeDtypeStruct((NW, EW), _f32),
    mesh=_mesh,
    compiler_params=pltpu.CompilerParams(needs_layout_passes=False),
    scratch_types=[
        pltpu.VMEM((EW,), _i32),
        pltpu.VMEM((EW,), _i32),
        pltpu.VMEM((EW,), _f32),
        pltpu.VMEM((NP,), _f32),
        pltpu.VMEM((EW,), _f32),
    ],
)
def _wgt_call(a_hbm, b_hbm, dv_hbm, dinv_hbm, out_hbm, av, bv, dvv, dinv_v, wv):
    c = lax.axis_index("c")
    s = lax.axis_index("s")
    wid = s * NC + c
    pltpu.sync_copy(dinv_hbm, dinv_v)
    pltpu.sync_copy(a_hbm.at[wid], av)
    pltpu.sync_copy(b_hbm.at[wid], bv)
    pltpu.sync_copy(dv_hbm.at[wid], dvv)

    @pl.loop(0, EW // 16)
    def _(k):
        sl = pl.ds(k * 16, 16)
        g1 = plsc.load_gather(dinv_v, [av[sl]])
        g2 = plsc.load_gather(dinv_v, [bv[sl]])
        d16 = dvv[sl]
        wv[sl] = g1 * g2 * jnp.exp(-d16 * d16)

    pltpu.sync_copy(wv, out_hbm.at[wid])


KQ = 64
NBQ = 320
CB = 16
NO = NBQ // CB
CBH = CB // 2
CHP = NBQ * KQ


@functools.partial(
    pl.kernel,
    out_type=jax.ShapeDtypeStruct((NC, NP, D), _f32),
    mesh=_mesh,
    compiler_params=pltpu.CompilerParams(needs_layout_passes=False),
    scratch_types=[
        pltpu.VMEM((CB, KQ), _i32),
        pltpu.VMEM((CB, KQ), _i32),
        pltpu.VMEM((CB, KQ), _f32),
        pltpu.VMEM((KQ, D), _f32),
        pltpu.VMEM((KQ, D), _f32),
        pltpu.VMEM((KQ, D), _f32),
        pltpu.VMEM((KQ, D), _f32),
        pltpu.VMEM_SHARED((NP, D), _f32),
        pltpu.SemaphoreType.DMA,
        pltpu.SemaphoreType.DMA,
        pltpu.SemaphoreType.DMA,
        pltpu.SemaphoreType.DMA,
    ],
)
def _spmm_call(n1_hbm, n2_hbm, w_hbm, enc_hbm, out_hbm,
               n1c, n2c, wc, gbuf_a, gbuf_b, sbuf_a, sbuf_b, acc_sh,
               gsem_a, gsem_b, ssem_a, ssem_b):
    c = lax.axis_index("c")
    s = lax.axis_index("s")
    wid = s * NC + c

    @pl.loop(0, KQ)
    def _(r):
        for jj in range(D // 16):
            gbuf_a[r, pl.ds(jj * 16, 16)] = jnp.zeros((16,), _f32)

    @pl.loop(0, RPT // KQ)
    def _(t):
        pltpu.sync_copy(gbuf_a, acc_sh.at[pl.ds(s * RPT + t * KQ, KQ)])

    plsc.subcore_barrier()

    def _scale(gbuf, sbuf, j):
        @pl.loop(0, KQ // 16)
        def _(g):
            w16 = wc[j, pl.ds(g * 16, 16)]
            for t in range(16):
                r = g * 16 + t
                wvec = jnp.full((16,), w16[t], _f32)
                for jj in range(D // 16):
                    sl = pl.ds(jj * 16, 16)
                    sbuf[r, sl] = gbuf[r, sl] * wvec

    def _wait_g(gbuf, sem):
        pltpu.make_async_copy(enc_hbm.at[n2c.at[0]], gbuf, sem).wait()

    def _wait_s(sbuf, sem):
        pltpu.make_async_copy(sbuf, acc_sh.at[n1c.at[0]], sem).wait()

    @pl.loop(0, NO)
    def _(o):
        pltpu.sync_copy(n1_hbm.at[wid, o], n1c)
        pltpu.sync_copy(n2_hbm.at[wid, o], n2c)
        pltpu.sync_copy(w_hbm.at[wid, o], wc)

        pltpu.async_copy(enc_hbm.at[n2c.at[0]], gbuf_a, gsem_a)
        pltpu.async_copy(enc_hbm.at[n2c.at[1]], gbuf_b, gsem_b)

        @pl.loop(0, CBH)
        def _(p):
            j0 = 2 * p
            j1 = 2 * p + 1
            _wait_g(gbuf_a, gsem_a)

            @pl.when(p > 0)
            def _():
                _wait_s(sbuf_a, ssem_a)

            _scale(gbuf_a, sbuf_a, j0)

            @pl.when(p < CBH - 1)
            def _():
                pltpu.async_copy(enc_hbm.at[n2c.at[j0 + 2]], gbuf_a, gsem_a)

            pltpu.async_copy(sbuf_a, acc_sh.at[n1c.at[j0]], ssem_a, add=True)

            _wait_g(gbuf_b, gsem_b)

            @pl.when(p > 0)
            def _():
                _wait_s(sbuf_b, ssem_b)

            _scale(gbuf_b, sbuf_b, j1)

            @pl.when(p < CBH - 1)
            def _():
                pltpu.async_copy(enc_hbm.at[n2c.at[j1 + 2]], gbuf_b, gsem_b)

            pltpu.async_copy(sbuf_b, acc_sh.at[n1c.at[j1]], ssem_b, add=True)

        _wait_s(sbuf_a, ssem_a)
        _wait_s(sbuf_b, ssem_b)

    plsc.subcore_barrier()
    pltpu.sync_copy(acc_sh.at[pl.ds(s * RPT, RPT)],
                    out_hbm.at[c, pl.ds(s * RPT, RPT)])


@functools.partial(
    pl.kernel,
    out_type=jax.ShapeDtypeStruct((NW * G, D), _f32),
    mesh=_mesh,
    scratch_types=[
        pltpu.VMEM((G,), _i32),
        pltpu.VMEM((G, D), _f32),
        pltpu.SemaphoreType.DMA,
    ],
)
def _gather_call(idx_hbm, enc_hbm, out_hbm, idxv, rows_v, sem):
    c = lax.axis_index("c")
    s = lax.axis_index("s")
    wid = s * NC + c
    pltpu.sync_copy(idx_hbm.at[wid], idxv)
    pltpu.async_copy(enc_hbm.at[idxv], rows_v, sem).wait()
    pltpu.sync_copy(rows_v, out_hbm.at[pl.ds(wid * G, G)])


def _dinv_body(degp_ref, dinv_ref, dinv2_ref):
    deg = degp_ref[0] + degp_ref[1] + 1.0
    dinv_ref[...] = lax.rsqrt(deg)
    dinv2_ref[...] = 1.0 / deg


_dinv_call = pl.pallas_call(
    _dinv_body,
    out_shape=(jax.ShapeDtypeStruct((NP // D, D), _f32),
               jax.ShapeDtypeStruct((NP // D, D), _f32)),
)


RB = 512


def _xform_body(a0_ref, a1_ref, enc_ref, d2_ref, w_ref, b_ref, out_ref):
    x = a0_ref[...] + a1_ref[...] + d2_ref[...] * enc_ref[...]
    msg = lax.dot_general(x, w_ref[...], (((1,), (1,)), ((), ())),
                          preferred_element_type=_f32) + b_ref[...]
    act = jnp.where(msg >= 0, msg, 0.01 * msg)
    nrm = jnp.sqrt(jnp.sum(act * act, axis=1, keepdims=True))
    out_ref[...] = act / jnp.maximum(nrm, 1e-12)


_xform_call = pl.pallas_call(
    _xform_body,
    grid=(NP // RB,),
    in_specs=[
        pl.BlockSpec((RB, D), lambda i: (i, 0)),
        pl.BlockSpec((RB, D), lambda i: (i, 0)),
        pl.BlockSpec((RB, D), lambda i: (i, 0)),
        pl.BlockSpec((RB, 1), lambda i: (i, 0)),
        pl.BlockSpec((D, D), lambda i: (0, 0)),
        pl.BlockSpec((1, D), lambda i: (0, 0)),
    ],
    out_specs=pl.BlockSpec((RB, D), lambda i: (i, 0)),
    out_shape=jax.ShapeDtypeStruct((NP, D), _f32),
)


def _attn_body(seq_ref, wqkv_ref, bqkv_ref, wo_ref, bo_ref, out_ref):
    sq = seq_ref[0]
    qkv = lax.dot_general(sq, wqkv_ref[...], (((1,), (1,)), ((), ())),
                          preferred_element_type=_f32) + bqkv_ref[...]
    kmask = lax.broadcasted_iota(jnp.int32, (LP, LP), 1) < L
    heads = []
    for h in range(8):
        qh = qkv[:, h * 16:(h + 1) * 16]
        kh = qkv[:, D + h * 16:D + (h + 1) * 16]
        vh = qkv[:, 2 * D + h * 16:2 * D + (h + 1) * 16]
        sc = lax.dot_general(qh, kh, (((1,), (1,)), ((), ())),
                             preferred_element_type=_f32) * 0.25
        sc = jnp.where(kmask, sc, -1e30)
        m = jnp.max(sc, axis=1, keepdims=True)
        p = jnp.exp(sc - m)
        p = p / jnp.sum(p, axis=1, keepdims=True)
        heads.append(lax.dot_general(p, vh, (((1,), (0,)), ((), ())),
                                     preferred_element_type=_f32))
    o = jnp.concatenate(heads, axis=1)
    ao = lax.dot_general(o, wo_ref[...], (((1,), (1,)), ((), ())),
                         preferred_element_type=_f32) + bo_ref[...]
    rmask = lax.broadcasted_iota(jnp.int32, (LP, 1), 0) < L
    out_ref[...] = (jnp.sum(jnp.where(rmask, ao, 0.0), axis=0,
                            keepdims=True) / float(L)).reshape(1, 1, D)


_attn_call = pl.pallas_call(
    _attn_body,
    grid=(B,),
    in_specs=[
        pl.BlockSpec((1, LP, D), lambda i: (i, 0, 0)),
        pl.BlockSpec((3 * D, D), lambda i: (0, 0)),
        pl.BlockSpec((1, 3 * D), lambda i: (0, 0)),
        pl.BlockSpec((D, D), lambda i: (0, 0)),
        pl.BlockSpec((1, D), lambda i: (0, 0)),
    ],
    out_specs=pl.BlockSpec((1, 1, D), lambda i: (i, 0, 0)),
    out_shape=jax.ShapeDtypeStruct((B, 1, D), _f32),
)


def kernel(poi_embeds_weight, dist_edges, dist_vec, data_poi, data_x,
           data_batch, W0, b0, W1, b1, in_proj_w, in_proj_b, out_w, out_b):
    a = dist_edges[0].astype(_i32)
    b_ = dist_edges[1].astype(_i32)
    epad = jnp.zeros((NW * CHP - NE,), _i32)
    n1 = jnp.concatenate([a, b_, epad]).reshape(NW, NO, CB, KQ)
    n2 = jnp.concatenate([b_, a, epad]).reshape(NW, NO, CB, KQ)

    degp = _deg_call(jnp.concatenate([a, b_]).reshape(NW, NB, K))
    dinv, dinv2 = _dinv_call(degp.reshape(NC, NP // D, D))
    dinv_flat = dinv.reshape(NP)

    w = _wgt_call(a.reshape(NW, EW), b_.reshape(NW, EW),
                  dist_vec.astype(_f32).reshape(NW, EW), dinv_flat)
    wflat = w.reshape(E)
    w2 = jnp.concatenate([wflat, wflat,
                          jnp.zeros((NW * CHP - NE,), _f32)]
                         ).reshape(NW, NO, CB, KQ)

    enc0 = jnp.pad(poi_embeds_weight.astype(_f32), ((0, NP - N), (0, 0)))
    d2col = dinv2.reshape(NP, 1)

    acc = _spmm_call(n1, n2, w2, enc0)
    enc1 = _xform_call(acc[0], acc[1], enc0, d2col, W0, b0.reshape(1, D))
    acc2 = _spmm_call(n1, n2, w2, enc1)
    enc2 = _xform_call(acc2[0], acc2[1], enc1, d2col, W1, b1.reshape(1, D))

    idx_all = jnp.concatenate([
        jnp.pad(data_x.astype(_i32).reshape(B, L), ((0, 0), (0, LP - L))
                ).reshape(-1),
        data_poi.astype(_i32),
        jnp.zeros((NW * G - B * LP - B,), _i32),
    ]).reshape(NW, G)
    rows = _gather_call(idx_all, enc2)
    seq = rows[:B * LP].reshape(B, LP, D)
    poi_embed = rows[B * LP:B * LP + B]

    aggr = _attn_call(seq, in_proj_w, in_proj_b.reshape(1, 3 * D),
                      out_w, out_b.reshape(1, D)).reshape(B, D)
    return (aggr, poi_embed)


# speedup vs baseline: 2.5063x; 2.5063x over previous
import functools

import jax
import jax.numpy as jnp
from jax import lax
from jax.experimental import pallas as pl
from jax.experimental.pallas import tpu as pltpu
from jax.experimental.pallas import tpu_sc as plsc

N = 10000
NP = 10240
D = 128
E = 320000
NE = 2 * E
B = 16
L = 50
LP = 64
NC = 2
NS = 16
NW = NC * NS
K = 80
CH = NE // NW
NB = CH // K
EW = E // NW
RPT = NP // NS
G = 1280 // NW

_f32 = jnp.float32
_i32 = jnp.int32

_mesh = plsc.VectorSubcoreMesh(
    core_axis_name="c", subcore_axis_name="s", num_cores=NC, num_subcores=NS)


@functools.partial(
    pl.kernel,
    out_type=jax.ShapeDtypeStruct((NC, NP), _f32),
    mesh=_mesh,
    scratch_types=[
        pltpu.VMEM((NB, K), _i32),
        pltpu.VMEM((K,), _f32),
        pltpu.VMEM((RPT,), _f32),
        pltpu.VMEM_SHARED((NP,), _f32),
    ],
)
def _deg_call(n1_hbm, out_hbm, idx_v, ones_v, zrow_v, deg_sh):
    c = lax.axis_index("c")
    s = lax.axis_index("s")
    wid = s * NC + c
    pltpu.sync_copy(n1_hbm.at[wid], idx_v)

    @pl.loop(0, K // 16)
    def _(i):
        ones_v[pl.ds(i * 16, 16)] = jnp.ones((16,), _f32)

    @pl.loop(0, RPT // 16)
    def _(i):
        zrow_v[pl.ds(i * 16, 16)] = jnp.zeros((16,), _f32)

    pltpu.sync_copy(zrow_v, deg_sh.at[pl.ds(s * RPT, RPT)])
    plsc.subcore_barrier()

    @pl.loop(0, NB)
    def _(j):
        pltpu.sync_copy(ones_v, deg_sh.at[idx_v.at[j]], add=True)

    plsc.subcore_barrier()
    pltpu.sync_copy(deg_sh.at[pl.ds(s * RPT, RPT)],
                    out_hbm.at[c, pl.ds(s * RPT, RPT)])


@functools.partial(
    pl.kernel,
    out_type=jax.ShapeDtypeStruct((NW, EW), _f32),
    mesh=_mesh,
    compiler_params=pltpu.CompilerParams(needs_layout_passes=False),
    scratch_types=[
        pltpu.VMEM((EW,), _i32),
        pltpu.VMEM((EW,), _i32),
        pltpu.VMEM((EW,), _f32),
        pltpu.VMEM((NP,), _f32),
        pltpu.VMEM((EW,), _f32),
    ],
)
def _wgt_call(a_hbm, b_hbm, dv_hbm, dinv_hbm, out_hbm, av, bv, dvv, dinv_v, wv):
    c = lax.axis_index("c")
    s = lax.axis_index("s")
    wid = s * NC + c
    pltpu.sync_copy(dinv_hbm, dinv_v)
    pltpu.sync_copy(a_hbm.at[wid], av)
    pltpu.sync_copy(b_hbm.at[wid], bv)
    pltpu.sync_copy(dv_hbm.at[wid], dvv)

    @pl.loop(0, EW // 16)
    def _(k):
        sl = pl.ds(k * 16, 16)
        g1 = plsc.load_gather(dinv_v, [av[sl]])
        g2 = plsc.load_gather(dinv_v, [bv[sl]])
        d16 = dvv[sl]
        wv[sl] = g1 * g2 * jnp.exp(-d16 * d16)

    pltpu.sync_copy(wv, out_hbm.at[wid])


CB = 50
NO = NB // CB
CBH = CB // 2


@functools.partial(
    pl.kernel,
    out_type=jax.ShapeDtypeStruct((NC, NP, D), _f32),
    mesh=_mesh,
    compiler_params=pltpu.CompilerParams(needs_layout_passes=False),
    scratch_types=[
        pltpu.VMEM((CB, K), _i32),
        pltpu.VMEM((CB, K), _i32),
        pltpu.VMEM((CB, K), _f32),
        pltpu.VMEM((K, D), _f32),
        pltpu.VMEM((K, D), _f32),
        pltpu.VMEM_SHARED((NP, D), _f32),
        pltpu.SemaphoreType.DMA,
        pltpu.SemaphoreType.DMA,
        pltpu.SemaphoreType.DMA,
        pltpu.SemaphoreType.DMA,
    ],
)
def _spmm_call(n1_hbm, n2_hbm, w_hbm, enc_hbm, out_hbm,
               n1c, n2c, wc, rows_a, rows_b, acc_sh,
               gsem_a, gsem_b, ssem_a, ssem_b):
    c = lax.axis_index("c")
    s = lax.axis_index("s")
    wid = s * NC + c

    @pl.loop(0, K)
    def _(r):
        for jj in range(D // 16):
            rows_a[r, pl.ds(jj * 16, 16)] = jnp.zeros((16,), _f32)

    @pl.loop(0, RPT // K)
    def _(t):
        pltpu.sync_copy(rows_a, acc_sh.at[pl.ds(s * RPT + t * K, K)])

    plsc.subcore_barrier()

    def _scale(rows_v, j):
        @pl.loop(0, K // 16)
        def _(g):
            w16 = wc[j, pl.ds(g * 16, 16)]
            for t in range(16):
                r = g * 16 + t
                wvec = jnp.full((16,), w16[t], _f32)
                for jj in range(D // 16):
                    sl = pl.ds(jj * 16, 16)
                    rows_v[r, sl] = rows_v[r, sl] * wvec

    @pl.loop(0, NO)
    def _(o):
        pltpu.sync_copy(n1_hbm.at[wid, o], n1c)
        pltpu.sync_copy(n2_hbm.at[wid, o], n2c)
        pltpu.sync_copy(w_hbm.at[wid, o], wc)

        pltpu.async_copy(enc_hbm.at[n2c.at[0]], rows_a, gsem_a)
        pltpu.async_copy(enc_hbm.at[n2c.at[1]], rows_b, gsem_b)

        def _wait_g(rows_v, sem):
            pltpu.make_async_copy(enc_hbm.at[n2c.at[0]], rows_v, sem).wait()

        def _wait_s(rows_v, sem):
            pltpu.make_async_copy(rows_v, acc_sh.at[n1c.at[0]], sem).wait()

        @pl.loop(0, CBH)
        def _(p):
            j0 = 2 * p
            j1 = 2 * p + 1
            _wait_g(rows_a, gsem_a)
            _scale(rows_a, j0)
            pltpu.async_copy(rows_a, acc_sh.at[n1c.at[j0]], ssem_a, add=True)
            _wait_g(rows_b, gsem_b)
            _scale(rows_b, j1)
            pltpu.async_copy(rows_b, acc_sh.at[n1c.at[j1]], ssem_b, add=True)

            @pl.when(p < CBH - 1)
            def _():
                _wait_s(rows_a, ssem_a)
                pltpu.async_copy(enc_hbm.at[n2c.at[j0 + 2]], rows_a, gsem_a)
                _wait_s(rows_b, ssem_b)
                pltpu.async_copy(enc_hbm.at[n2c.at[j1 + 2]], rows_b, gsem_b)

        _wait_s(rows_a, ssem_a)
        _wait_s(rows_b, ssem_b)

    plsc.subcore_barrier()
    pltpu.sync_copy(acc_sh.at[pl.ds(s * RPT, RPT)],
                    out_hbm.at[c, pl.ds(s * RPT, RPT)])


@functools.partial(
    pl.kernel,
    out_type=jax.ShapeDtypeStruct((NW * G, D), _f32),
    mesh=_mesh,
    scratch_types=[
        pltpu.VMEM((G,), _i32),
        pltpu.VMEM((G, D), _f32),
        pltpu.SemaphoreType.DMA,
    ],
)
def _gather_call(idx_hbm, enc_hbm, out_hbm, idxv, rows_v, sem):
    c = lax.axis_index("c")
    s = lax.axis_index("s")
    wid = s * NC + c
    pltpu.sync_copy(idx_hbm.at[wid], idxv)
    pltpu.async_copy(enc_hbm.at[idxv], rows_v, sem).wait()
    pltpu.sync_copy(rows_v, out_hbm.at[pl.ds(wid * G, G)])


def _dinv_body(degp_ref, dinv_ref, dinv2_ref):
    deg = degp_ref[0] + degp_ref[1] + 1.0
    dinv_ref[...] = lax.rsqrt(deg)
    dinv2_ref[...] = 1.0 / deg


_dinv_call = pl.pallas_call(
    _dinv_body,
    out_shape=(jax.ShapeDtypeStruct((NP // D, D), _f32),
               jax.ShapeDtypeStruct((NP // D, D), _f32)),
)


RB = 512


def _xform_body(a0_ref, a1_ref, enc_ref, d2_ref, w_ref, b_ref, out_ref):
    x = a0_ref[...] + a1_ref[...] + d2_ref[...] * enc_ref[...]
    msg = lax.dot_general(x, w_ref[...], (((1,), (1,)), ((), ())),
                          preferred_element_type=_f32) + b_ref[...]
    act = jnp.where(msg >= 0, msg, 0.01 * msg)
    nrm = jnp.sqrt(jnp.sum(act * act, axis=1, keepdims=True))
    out_ref[...] = act / jnp.maximum(nrm, 1e-12)


_xform_call = pl.pallas_call(
    _xform_body,
    grid=(NP // RB,),
    in_specs=[
        pl.BlockSpec((RB, D), lambda i: (i, 0)),
        pl.BlockSpec((RB, D), lambda i: (i, 0)),
        pl.BlockSpec((RB, D), lambda i: (i, 0)),
        pl.BlockSpec((RB, 1), lambda i: (i, 0)),
        pl.BlockSpec((D, D), lambda i: (0, 0)),
        pl.BlockSpec((1, D), lambda i: (0, 0)),
    ],
    out_specs=pl.BlockSpec((RB, D), lambda i: (i, 0)),
    out_shape=jax.ShapeDtypeStruct((NP, D), _f32),
)


def _attn_body(seq_ref, wqkv_ref, bqkv_ref, wo_ref, bo_ref, out_ref):
    sq = seq_ref[0]
    qkv = lax.dot_general(sq, wqkv_ref[...], (((1,), (1,)), ((), ())),
                          preferred_element_type=_f32) + bqkv_ref[...]
    kmask = lax.broadcasted_iota(jnp.int32, (LP, LP), 1) < L
    heads = []
    for h in range(8):
        qh = qkv[:, h * 16:(h + 1) * 16]
        kh = qkv[:, D + h * 16:D + (h + 1) * 16]
        vh = qkv[:, 2 * D + h * 16:2 * D + (h + 1) * 16]
        sc = lax.dot_general(qh, kh, (((1,), (1,)), ((), ())),
                             preferred_element_type=_f32) * 0.25
        sc = jnp.where(kmask, sc, -1e30)
        m = jnp.max(sc, axis=1, keepdims=True)
        p = jnp.exp(sc - m)
        p = p / jnp.sum(p, axis=1, keepdims=True)
        heads.append(lax.dot_general(p, vh, (((1,), (0,)), ((), ())),
                                     preferred_element_type=_f32))
    o = jnp.concatenate(heads, axis=1)
    ao = lax.dot_general(o, wo_ref[...], (((1,), (1,)), ((), ())),
                         preferred_element_type=_f32) + bo_ref[...]
    rmask = lax.broadcasted_iota(jnp.int32, (LP, 1), 0) < L
    out_ref[...] = (jnp.sum(jnp.where(rmask, ao, 0.0), axis=0,
                            keepdims=True) / float(L)).reshape(1, 1, D)


_attn_call = pl.pallas_call(
    _attn_body,
    grid=(B,),
    in_specs=[
        pl.BlockSpec((1, LP, D), lambda i: (i, 0, 0)),
        pl.BlockSpec((3 * D, D), lambda i: (0, 0)),
        pl.BlockSpec((1, 3 * D), lambda i: (0, 0)),
        pl.BlockSpec((D, D), lambda i: (0, 0)),
        pl.BlockSpec((1, D), lambda i: (0, 0)),
    ],
    out_specs=pl.BlockSpec((1, 1, D), lambda i: (i, 0, 0)),
    out_shape=jax.ShapeDtypeStruct((B, 1, D), _f32),
)


def kernel(poi_embeds_weight, dist_edges, dist_vec, data_poi, data_x,
           data_batch, W0, b0, W1, b1, in_proj_w, in_proj_b, out_w, out_b):
    a = dist_edges[0].astype(_i32)
    b_ = dist_edges[1].astype(_i32)
    n1 = jnp.concatenate([a, b_]).reshape(NW, NO, CB, K)
    n2 = jnp.concatenate([b_, a]).reshape(NW, NO, CB, K)

    degp = _deg_call(n1.reshape(NW, NB, K))
    dinv, dinv2 = _dinv_call(degp.reshape(NC, NP // D, D))
    dinv_flat = dinv.reshape(NP)

    w = _wgt_call(a.reshape(NW, EW), b_.reshape(NW, EW),
                  dist_vec.astype(_f32).reshape(NW, EW), dinv_flat)
    wflat = w.reshape(E)
    w2 = jnp.concatenate([wflat, wflat]).reshape(NW, NO, CB, K)

    enc0 = jnp.pad(poi_embeds_weight.astype(_f32), ((0, NP - N), (0, 0)))
    d2col = dinv2.reshape(NP, 1)

    acc = _spmm_call(n1, n2, w2, enc0)
    enc1 = _xform_call(acc[0], acc[1], enc0, d2col, W0, b0.reshape(1, D))
    acc2 = _spmm_call(n1, n2, w2, enc1)
    enc2 = _xform_call(acc2[0], acc2[1], enc1, d2col, W1, b1.reshape(1, D))

    idx_all = jnp.concatenate([
        jnp.pad(data_x.astype(_i32).reshape(B, L), ((0, 0), (0, LP - L))
                ).reshape(-1),
        data_poi.astype(_i32),
        jnp.zeros((NW * G - B * LP - B,), _i32),
    ]).reshape(NW, G)
    rows = _gather_call(idx_all, enc2)
    seq = rows[:B * LP].reshape(B, LP, D)
    poi_embed = rows[B * LP:B * LP + B]

    aggr = _attn_call(seq, in_proj_w, in_proj_b.reshape(1, 3 * D),
                      out_w, out_b.reshape(1, D)).reshape(B, D)
    return (aggr, poi_embed)
